# baseline (speedup 1.0000x reference)
import functools

import jax
import jax.numpy as jnp
from jax import lax
from jax.experimental import pallas as pl
from jax.experimental.pallas import tpu as pltpu
from jax.experimental.pallas import tpu_sc as plsc

NUM_DIM = 64
PACK = 128
LANES = 16
NC, NS = 2, 16
NW = NC * NS
CHUNK = 128
HALF = 256

_BITREV = [0, 8, 4, 12, 2, 10, 6, 14, 1, 9, 5, 13, 3, 11, 7, 15]


def _irt_body(theta_hbm, alpha_hbm, beta_hbm, sidx_hbm, qidx_hbm,
              spar_hbm, qpar_hbm, out_hbm,
              sidx_v, qidx_v, spar_v, qpar_v,
              theta_v, alpha_v, beta_v, out_v, sem0, sem1,
              *, rows_per_worker):
    wid = lax.axis_index("s") * NC + lax.axis_index("c")
    n_chunks = rows_per_worker // CHUNK
    sems = (sem0, sem1)

    pltpu.sync_copy(sidx_hbm.at[wid], sidx_v)
    pltpu.sync_copy(qidx_hbm.at[wid], qidx_v)
    pltpu.sync_copy(spar_hbm.at[wid], spar_v)
    pltpu.sync_copy(qpar_hbm.at[wid], qpar_v)

    lane = lax.iota(jnp.int32, LANES)
    perm_idx = {d: lane ^ d for d in (8, 4, 2, 1)}
    masks = {d: (lane & d) == 0 for d in (8, 4, 2, 1)}
    gather_dnums = lax.GatherDimensionNumbers(
        offset_dims=(), collapsed_slice_dims=(0,), start_index_map=(0,))

    def permute(v, idx):
        return lax.gather(v, idx[:, None], gather_dnums, slice_sizes=(1,),
                          mode=lax.GatherScatterMode.PROMISE_IN_BOUNDS)

    groups_per_chunk = CHUNK // LANES

    def fire(j):
        slot = j % 2
        descs = (pltpu.make_async_copy(
                     theta_hbm.at[sidx_v.at[j]], theta_v.at[slot], sems[slot]),
                 pltpu.make_async_copy(
                     alpha_hbm.at[qidx_v.at[j]], alpha_v.at[slot], sems[slot]),
                 pltpu.make_async_copy(
                     beta_hbm.at[qidx_v.at[j]], beta_v.at[slot], sems[slot]))
        for d in descs:
            d.start()
        return descs

    pending = fire(0)
    for j in range(n_chunks):
        nxt = fire(j + 1) if j + 1 < n_chunks else None
        for d in pending:
            d.wait()
        pending = nxt
        slot = j % 2

        def group(g, _, j=j, slot=slot):
            sp16 = spar_v[j, pl.ds(g * LANES, LANES)]
            qp16 = qpar_v[j, pl.ds(g * LANES, LANES)]
            base = g * LANES
            vs = []
            for k in range(LANES):
                r = base + _BITREV[k]
                bidx = jnp.full_like(lane, _BITREV[k])
                sp = permute(sp16, bidx)
                qp = permute(qp16, bidx)
                acc = None
                for c in range(NUM_DIM // LANES):
                    lo = pl.ds(c * LANES, LANES)
                    hi = pl.ds(NUM_DIM + c * LANES, LANES)
                    t_lo = theta_v[slot, r, lo]
                    t_ = t_lo + sp * (theta_v[slot, r, hi] - t_lo)
                    a_lo = alpha_v[slot, r, lo]
                    a_ = a_lo + qp * (alpha_v[slot, r, hi] - a_lo)
                    b_lo = beta_v[slot, r, lo]
                    b_ = b_lo + qp * (beta_v[slot, r, hi] - b_lo)
                    p = a_ * (t_ - b_)
                    acc = p if acc is None else acc + p
                vs.append(acc)
            for d in (8, 4, 2, 1):
                vs = [jnp.where(masks[d], a + permute(a, perm_idx[d]),
                                b + permute(b, perm_idx[d]))
                      for a, b in zip(vs[0::2], vs[1::2])]
            res = vs[0]
            out_v[pl.ds(j * CHUNK + base, LANES)] = (
                1.0 / (1.0 + jnp.exp(-res)))
            return ()

        lax.fori_loop(0, groups_per_chunk, group, ())

    pltpu.sync_copy(out_v, out_hbm.at[pl.ds(wid * rows_per_worker,
                                            rows_per_worker)])


_BLK = 20480


def _pack_split(n, blk=_BLK):
    nb = -(-(n // 2) // blk)
    s = nb * blk
    o = (-(-(n - s) // blk)) * blk
    assert o + (nb - 1) * blk < n and o + s >= n and o <= s
    return s, o


def _pack_body(x1_ref, x2_ref, out_ref):
    out_ref[:, 0:NUM_DIM] = x1_ref[...].T
    out_ref[:, NUM_DIM:PACK] = x2_ref[...].T


def _pack_table(table):
    n = table.shape[0]
    s, o = _pack_split(n)
    n_blocks = s // _BLK
    o_blocks = o // _BLK
    table_t = table.T
    return pl.pallas_call(
        _pack_body,
        grid=(n_blocks,),
        in_specs=[
            pl.BlockSpec((NUM_DIM, _BLK), lambda j: (0, j)),
            pl.BlockSpec((NUM_DIM, _BLK), lambda j, ob=o_blocks: (0, j + ob)),
        ],
        out_specs=pl.BlockSpec((_BLK, PACK), lambda j: (j, 0)),
        out_shape=jax.ShapeDtypeStruct((s, PACK), jnp.float32),
    )(table_t, table_t)


def _pack2_body(x1_ref, x2_ref, y1_ref, y2_ref, out1_ref, out2_ref):
    out1_ref[:, 0:NUM_DIM] = x1_ref[...].T
    out1_ref[:, NUM_DIM:PACK] = x2_ref[...].T
    out2_ref[:, 0:NUM_DIM] = y1_ref[...].T
    out2_ref[:, NUM_DIM:PACK] = y2_ref[...].T


_BLK2 = 8192


def _pack_table2(ta, tb):
    n = ta.shape[0]
    s, o = _pack_split(n, _BLK2)
    n_blocks = s // _BLK2
    o_blocks = o // _BLK2
    ta_t = ta.T
    tb_t = tb.T
    in_lo = pl.BlockSpec((NUM_DIM, _BLK2), lambda j: (0, j))
    in_hi = pl.BlockSpec((NUM_DIM, _BLK2), lambda j, ob=o_blocks: (0, j + ob))
    out_spec = pl.BlockSpec((_BLK2, PACK), lambda j: (j, 0))
    return pl.pallas_call(
        _pack2_body,
        grid=(n_blocks,),
        in_specs=[in_lo, in_hi, in_lo, in_hi],
        out_specs=[out_spec, out_spec],
        out_shape=[jax.ShapeDtypeStruct((s, PACK), jnp.float32)] * 2,
    )(ta_t, ta_t, tb_t, tb_t)


@jax.jit
def kernel(student_ids, question_ids, theta_table, alpha_table, beta_table):
    batch = student_ids.shape[0]
    rows_per_worker = batch // NW
    n_chunks = rows_per_worker // CHUNK
    s_split, s_off = _pack_split(theta_table.shape[0])
    q_split, q_off = _pack_split(alpha_table.shape[0], _BLK2)

    sid = student_ids.astype(jnp.int32)
    qid = question_ids.astype(jnp.int32)
    sidx = jnp.where(sid < s_split, sid, sid - s_off)
    qidx = jnp.where(qid < q_split, qid, qid - q_off)
    sidx = sidx.reshape(NW, n_chunks, CHUNK)
    qidx = qidx.reshape(NW, n_chunks, CHUNK)
    spar = (sid >= s_split).astype(jnp.float32).reshape(NW, n_chunks, CHUNK)
    qpar = (qid >= q_split).astype(jnp.float32).reshape(NW, n_chunks, CHUNK)

    theta2 = _pack_table(theta_table)
    alpha2, beta2 = _pack_table2(alpha_table, beta_table)

    mesh = plsc.VectorSubcoreMesh(core_axis_name="c", subcore_axis_name="s")
    body = functools.partial(_irt_body, rows_per_worker=rows_per_worker)
    run = pl.kernel(
        body,
        mesh=mesh,
        compiler_params=pltpu.CompilerParams(use_tc_tiling_on_sc=True),
        out_type=jax.ShapeDtypeStruct((batch,), jnp.float32),
        scratch_types=[
            pltpu.VMEM((n_chunks, CHUNK), jnp.int32),
            pltpu.VMEM((n_chunks, CHUNK), jnp.int32),
            pltpu.VMEM((n_chunks, CHUNK), jnp.float32),
            pltpu.VMEM((n_chunks, CHUNK), jnp.float32),
            pltpu.VMEM((2, CHUNK, PACK), jnp.float32),
            pltpu.VMEM((2, CHUNK, PACK), jnp.float32),
            pltpu.VMEM((2, CHUNK, PACK), jnp.float32),
            pltpu.VMEM((rows_per_worker,), jnp.float32),
            pltpu.SemaphoreType.DMA,
            pltpu.SemaphoreType.DMA,
        ],
    )
    pred = run(theta2, alpha2, beta2, sidx, qidx, spar, qpar)
    return pred.reshape(batch, 1)


# speedup vs baseline: 1.0061x; 1.0061x over previous
import functools

import jax
import jax.numpy as jnp
from jax import lax
from jax.experimental import pallas as pl
from jax.experimental.pallas import tpu as pltpu
from jax.experimental.pallas import tpu_sc as plsc

NUM_DIM = 64
PACK = 128
LANES = 16
NC, NS = 2, 16
NW = NC * NS
CHUNK = 128
HALF = 256

_BITREV = [0, 8, 4, 12, 2, 10, 6, 14, 1, 9, 5, 13, 3, 11, 7, 15]


def _irt_body(theta_hbm, alpha_hbm, beta_hbm, sidx_hbm, qidx_hbm,
              spar_hbm, qpar_hbm, out_hbm,
              sidx_v, qidx_v, spar_v, qpar_v,
              theta_v, alpha_v, beta_v, out_v, sem0, sem1,
              *, rows_per_worker):
    wid = lax.axis_index("s") * NC + lax.axis_index("c")
    n_chunks = rows_per_worker // CHUNK
    sems = (sem0, sem1)

    pltpu.sync_copy(sidx_hbm.at[wid], sidx_v)
    pltpu.sync_copy(qidx_hbm.at[wid], qidx_v)
    pltpu.sync_copy(spar_hbm.at[wid], spar_v)
    pltpu.sync_copy(qpar_hbm.at[wid], qpar_v)

    lane = lax.iota(jnp.int32, LANES)
    perm_idx = {d: lane ^ d for d in (8, 4, 2, 1)}
    masks = {d: (lane & d) == 0 for d in (8, 4, 2, 1)}
    gather_dnums = lax.GatherDimensionNumbers(
        offset_dims=(), collapsed_slice_dims=(0,), start_index_map=(0,))

    def permute(v, idx):
        return lax.gather(v, idx[:, None], gather_dnums, slice_sizes=(1,),
                          mode=lax.GatherScatterMode.PROMISE_IN_BOUNDS)

    groups_per_chunk = CHUNK // LANES

    def fire(j):
        slot = j % 2
        descs = (pltpu.make_async_copy(
                     theta_hbm.at[sidx_v.at[j]], theta_v.at[slot], sems[slot]),
                 pltpu.make_async_copy(
                     alpha_hbm.at[qidx_v.at[j]], alpha_v.at[slot], sems[slot]),
                 pltpu.make_async_copy(
                     beta_hbm.at[qidx_v.at[j]], beta_v.at[slot], sems[slot]))
        for d in descs:
            d.start()
        return descs

    pending = fire(0)
    for j in range(n_chunks):
        nxt = fire(j + 1) if j + 1 < n_chunks else None
        for d in pending:
            d.wait()
        pending = nxt
        slot = j % 2

        def group(g, _, j=j, slot=slot):
            sp16 = spar_v[j, pl.ds(g * LANES, LANES)]
            qp16 = qpar_v[j, pl.ds(g * LANES, LANES)]
            base = g * LANES
            vs = []
            for k in range(LANES):
                r = base + _BITREV[k]
                bidx = jnp.full_like(lane, _BITREV[k])
                sp = permute(sp16, bidx)
                qp = permute(qp16, bidx)
                acc = None
                for c in range(NUM_DIM // LANES):
                    lo = pl.ds(c * LANES, LANES)
                    hi = pl.ds(NUM_DIM + c * LANES, LANES)
                    t_lo = theta_v[slot, r, lo]
                    t_ = t_lo + sp * (theta_v[slot, r, hi] - t_lo)
                    a_lo = alpha_v[slot, r, lo]
                    a_ = a_lo + qp * (alpha_v[slot, r, hi] - a_lo)
                    b_lo = beta_v[slot, r, lo]
                    b_ = b_lo + qp * (beta_v[slot, r, hi] - b_lo)
                    p = a_ * (t_ - b_)
                    acc = p if acc is None else acc + p
                vs.append(acc)
            for d in (8, 4, 2, 1):
                vs = [jnp.where(masks[d], a + permute(a, perm_idx[d]),
                                b + permute(b, perm_idx[d]))
                      for a, b in zip(vs[0::2], vs[1::2])]
            res = vs[0]
            out_v[pl.ds(j * CHUNK + base, LANES)] = (
                1.0 / (1.0 + jnp.exp(-res)))
            return ()

        lax.fori_loop(0, groups_per_chunk, group, ())

    pltpu.sync_copy(out_v, out_hbm.at[pl.ds(wid * rows_per_worker,
                                            rows_per_worker)])


_BLK = 16384


def _pack_split(n, blk=_BLK):
    nb = -(-(n // 2) // blk)
    s = nb * blk
    o = (-(-(n - s) // blk)) * blk
    assert o + (nb - 1) * blk < n and o + s >= n and o <= s
    return s, o


def _pack_body(x1_ref, x2_ref, out_ref):
    out_ref[:, 0:NUM_DIM] = x1_ref[...].T
    out_ref[:, NUM_DIM:PACK] = x2_ref[...].T


def _pack_table(table):
    n = table.shape[0]
    s, o = _pack_split(n)
    n_blocks = s // _BLK
    o_blocks = o // _BLK
    table_t = table.T
    return pl.pallas_call(
        _pack_body,
        grid=(n_blocks,),
        in_specs=[
            pl.BlockSpec((NUM_DIM, _BLK), lambda j: (0, j)),
            pl.BlockSpec((NUM_DIM, _BLK), lambda j, ob=o_blocks: (0, j + ob)),
        ],
        out_specs=pl.BlockSpec((_BLK, PACK), lambda j: (j, 0)),
        out_shape=jax.ShapeDtypeStruct((s, PACK), jnp.float32),
    )(table_t, table_t)


def _pack2_body(x1_ref, x2_ref, y1_ref, y2_ref, out1_ref, out2_ref):
    out1_ref[:, 0:NUM_DIM] = x1_ref[...].T
    out1_ref[:, NUM_DIM:PACK] = x2_ref[...].T
    out2_ref[:, 0:NUM_DIM] = y1_ref[...].T
    out2_ref[:, NUM_DIM:PACK] = y2_ref[...].T


_BLK2 = 8192


def _pack_table2(ta, tb):
    n = ta.shape[0]
    s, o = _pack_split(n, _BLK2)
    n_blocks = s // _BLK2
    o_blocks = o // _BLK2
    ta_t = ta.T
    tb_t = tb.T
    in_lo = pl.BlockSpec((NUM_DIM, _BLK2), lambda j: (0, j))
    in_hi = pl.BlockSpec((NUM_DIM, _BLK2), lambda j, ob=o_blocks: (0, j + ob))
    out_spec = pl.BlockSpec((_BLK2, PACK), lambda j: (j, 0))
    return pl.pallas_call(
        _pack2_body,
        grid=(n_blocks,),
        in_specs=[in_lo, in_hi, in_lo, in_hi],
        out_specs=[out_spec, out_spec],
        out_shape=[jax.ShapeDtypeStruct((s, PACK), jnp.float32)] * 2,
    )(ta_t, ta_t, tb_t, tb_t)


@jax.jit
def kernel(student_ids, question_ids, theta_table, alpha_table, beta_table):
    batch = student_ids.shape[0]
    rows_per_worker = batch // NW
    n_chunks = rows_per_worker // CHUNK
    s_split, s_off = _pack_split(theta_table.shape[0])
    q_split, q_off = _pack_split(alpha_table.shape[0], _BLK2)

    sid = student_ids.astype(jnp.int32)
    qid = question_ids.astype(jnp.int32)
    sidx = jnp.where(sid < s_split, sid, sid - s_off)
    qidx = jnp.where(qid < q_split, qid, qid - q_off)
    sidx = sidx.reshape(NW, n_chunks, CHUNK)
    qidx = qidx.reshape(NW, n_chunks, CHUNK)
    spar = (sid >= s_split).astype(jnp.float32).reshape(NW, n_chunks, CHUNK)
    qpar = (qid >= q_split).astype(jnp.float32).reshape(NW, n_chunks, CHUNK)

    theta2 = _pack_table(theta_table)
    alpha2, beta2 = _pack_table2(alpha_table, beta_table)

    mesh = plsc.VectorSubcoreMesh(core_axis_name="c", subcore_axis_name="s")
    body = functools.partial(_irt_body, rows_per_worker=rows_per_worker)
    run = pl.kernel(
        body,
        mesh=mesh,
        compiler_params=pltpu.CompilerParams(use_tc_tiling_on_sc=True),
        out_type=jax.ShapeDtypeStruct((batch,), jnp.float32),
        scratch_types=[
            pltpu.VMEM((n_chunks, CHUNK), jnp.int32),
            pltpu.VMEM((n_chunks, CHUNK), jnp.int32),
            pltpu.VMEM((n_chunks, CHUNK), jnp.float32),
            pltpu.VMEM((n_chunks, CHUNK), jnp.float32),
            pltpu.VMEM((2, CHUNK, PACK), jnp.float32),
            pltpu.VMEM((2, CHUNK, PACK), jnp.float32),
            pltpu.VMEM((2, CHUNK, PACK), jnp.float32),
            pltpu.VMEM((rows_per_worker,), jnp.float32),
            pltpu.SemaphoreType.DMA,
            pltpu.SemaphoreType.DMA,
        ],
    )
    pred = run(theta2, alpha2, beta2, sidx, qidx, spar, qpar)
    return pred.reshape(batch, 1)


# speedup vs baseline: 1.0068x; 1.0007x over previous
import functools

import jax
import jax.numpy as jnp
from jax import lax
from jax.experimental import pallas as pl
from jax.experimental.pallas import tpu as pltpu
from jax.experimental.pallas import tpu_sc as plsc

NUM_DIM = 64
PACK = 128
LANES = 16
NC, NS = 2, 16
NW = NC * NS
CHUNK = 128

_BITREV = [0, 8, 4, 12, 2, 10, 6, 14, 1, 9, 5, 13, 3, 11, 7, 15]


def _irt_body(theta_hbm, alpha_hbm, beta_hbm, sidx_hbm, qidx_hbm,
              spar_hbm, qpar_hbm, out_hbm,
              sidx_v, qidx_v, spar_v, qpar_v,
              theta_v, alpha_v, beta_v, out_v, sem0, sem1,
              *, rows_per_worker):
    wid = lax.axis_index("s") * NC + lax.axis_index("c")
    n_chunks = rows_per_worker // CHUNK
    sems = (sem0, sem1)

    pltpu.sync_copy(sidx_hbm.at[wid], sidx_v)
    pltpu.sync_copy(qidx_hbm.at[wid], qidx_v)
    pltpu.sync_copy(spar_hbm.at[wid], spar_v)
    pltpu.sync_copy(qpar_hbm.at[wid], qpar_v)

    lane = lax.iota(jnp.int32, LANES)
    perm_idx = {d: lane ^ d for d in (8, 4, 2, 1)}
    masks = {d: (lane & d) == 0 for d in (8, 4, 2, 1)}
    gather_dnums = lax.GatherDimensionNumbers(
        offset_dims=(), collapsed_slice_dims=(0,), start_index_map=(0,))

    def permute(v, idx):
        return lax.gather(v, idx[:, None], gather_dnums, slice_sizes=(1,),
                          mode=lax.GatherScatterMode.PROMISE_IN_BOUNDS)

    groups_per_chunk = CHUNK // LANES

    def fire(j):
        slot = j % 2
        descs = (pltpu.make_async_copy(
                     theta_hbm.at[sidx_v.at[j]], theta_v.at[slot], sems[slot]),
                 pltpu.make_async_copy(
                     alpha_hbm.at[qidx_v.at[j]], alpha_v.at[slot], sems[slot]),
                 pltpu.make_async_copy(
                     beta_hbm.at[qidx_v.at[j]], beta_v.at[slot], sems[slot]))
        for d in descs:
            d.start()
        return descs

    pending = fire(0)
    for j in range(n_chunks):
        nxt = fire(j + 1) if j + 1 < n_chunks else None
        for d in pending:
            d.wait()
        pending = nxt
        slot = j % 2

        def group(g, _, j=j, slot=slot):
            sp16 = spar_v[j, pl.ds(g * LANES, LANES)]
            qp16 = qpar_v[j, pl.ds(g * LANES, LANES)]
            base = g * LANES
            vs = []
            for k in range(LANES):
                r = base + _BITREV[k]
                bidx = jnp.full_like(lane, _BITREV[k])
                sp = permute(sp16, bidx)
                qp = permute(qp16, bidx)
                acc = None
                for c in range(NUM_DIM // LANES):
                    lo = pl.ds(c * LANES, LANES)
                    hi = pl.ds(NUM_DIM + c * LANES, LANES)
                    t_lo = theta_v[slot, r, lo]
                    t_ = t_lo + sp * (theta_v[slot, r, hi] - t_lo)
                    a_lo = alpha_v[slot, r, lo]
                    a_ = a_lo + qp * (alpha_v[slot, r, hi] - a_lo)
                    b_lo = beta_v[slot, r, lo]
                    b_ = b_lo + qp * (beta_v[slot, r, hi] - b_lo)
                    p = a_ * (t_ - b_)
                    acc = p if acc is None else acc + p
                vs.append(acc)
            for d in (8, 4, 2, 1):
                vs = [jnp.where(masks[d], a + permute(a, perm_idx[d]),
                                b + permute(b, perm_idx[d]))
                      for a, b in zip(vs[0::2], vs[1::2])]
            res = vs[0]
            out_v[pl.ds(j * CHUNK + base, LANES)] = (
                1.0 / (1.0 + jnp.exp(-res)))
            return ()

        lax.fori_loop(0, groups_per_chunk, group, ())

    pltpu.sync_copy(out_v, out_hbm.at[pl.ds(wid * rows_per_worker,
                                            rows_per_worker)])


_BLK = 16384


def _pack_split(n, blk=_BLK):
    nb = -(-(n // 2) // blk)
    s = nb * blk
    o = (-(-(n - s) // blk)) * blk
    assert o + (nb - 1) * blk < n and o + s >= n and o <= s
    return s, o


def _pack_body(x1_ref, x2_ref, out_ref):
    out_ref[:, 0:NUM_DIM] = x1_ref[...].T
    out_ref[:, NUM_DIM:PACK] = x2_ref[...].T


def _pack_table(table):
    n = table.shape[0]
    s, o = _pack_split(n)
    n_blocks = s // _BLK
    o_blocks = o // _BLK
    table_t = table.T
    return pl.pallas_call(
        _pack_body,
        grid=(n_blocks,),
        in_specs=[
            pl.BlockSpec((NUM_DIM, _BLK), lambda j: (0, j)),
            pl.BlockSpec((NUM_DIM, _BLK), lambda j, ob=o_blocks: (0, j + ob)),
        ],
        out_specs=pl.BlockSpec((_BLK, PACK), lambda j: (j, 0)),
        out_shape=jax.ShapeDtypeStruct((s, PACK), jnp.float32),
    )(table_t, table_t)


def _pack2_body(x1_ref, x2_ref, y1_ref, y2_ref, out1_ref, out2_ref):
    out1_ref[:, 0:NUM_DIM] = x1_ref[...].T
    out1_ref[:, NUM_DIM:PACK] = x2_ref[...].T
    out2_ref[:, 0:NUM_DIM] = y1_ref[...].T
    out2_ref[:, NUM_DIM:PACK] = y2_ref[...].T


_BLK2 = 8192


def _pack_table2(ta, tb):
    n = ta.shape[0]
    s, o = _pack_split(n, _BLK2)
    n_blocks = s // _BLK2
    o_blocks = o // _BLK2
    ta_t = ta.T
    tb_t = tb.T
    in_lo = pl.BlockSpec((NUM_DIM, _BLK2), lambda j: (0, j))
    in_hi = pl.BlockSpec((NUM_DIM, _BLK2), lambda j, ob=o_blocks: (0, j + ob))
    out_spec = pl.BlockSpec((_BLK2, PACK), lambda j: (j, 0))
    return pl.pallas_call(
        _pack2_body,
        grid=(n_blocks,),
        in_specs=[in_lo, in_hi, in_lo, in_hi],
        out_specs=[out_spec, out_spec],
        out_shape=[jax.ShapeDtypeStruct((s, PACK), jnp.float32)] * 2,
    )(ta_t, ta_t, tb_t, tb_t)


@jax.jit
def kernel(student_ids, question_ids, theta_table, alpha_table, beta_table):
    batch = student_ids.shape[0]
    rows_per_worker = batch // NW
    n_chunks = rows_per_worker // CHUNK
    s_split, s_off = _pack_split(theta_table.shape[0])
    q_split, q_off = _pack_split(alpha_table.shape[0], _BLK2)

    sid = student_ids.astype(jnp.int32)
    qid = question_ids.astype(jnp.int32)
    sidx = jnp.where(sid < s_split, sid, sid - s_off)
    qidx = jnp.where(qid < q_split, qid, qid - q_off)
    sidx = sidx.reshape(NW, n_chunks, CHUNK)
    qidx = qidx.reshape(NW, n_chunks, CHUNK)
    spar = (sid >= s_split).astype(jnp.float32).reshape(NW, n_chunks, CHUNK)
    qpar = (qid >= q_split).astype(jnp.float32).reshape(NW, n_chunks, CHUNK)

    theta2 = _pack_table(theta_table)
    alpha2, beta2 = _pack_table2(alpha_table, beta_table)

    mesh = plsc.VectorSubcoreMesh(core_axis_name="c", subcore_axis_name="s")
    body = functools.partial(_irt_body, rows_per_worker=rows_per_worker)
    run = pl.kernel(
        body,
        mesh=mesh,
        compiler_params=pltpu.CompilerParams(use_tc_tiling_on_sc=True),
        out_type=jax.ShapeDtypeStruct((batch,), jnp.float32),
        scratch_types=[
            pltpu.VMEM((n_chunks, CHUNK), jnp.int32),
            pltpu.VMEM((n_chunks, CHUNK), jnp.int32),
            pltpu.VMEM((n_chunks, CHUNK), jnp.float32),
            pltpu.VMEM((n_chunks, CHUNK), jnp.float32),
            pltpu.VMEM((2, CHUNK, PACK), jnp.float32),
            pltpu.VMEM((2, CHUNK, PACK), jnp.float32),
            pltpu.VMEM((2, CHUNK, PACK), jnp.float32),
            pltpu.VMEM((rows_per_worker,), jnp.float32),
            pltpu.SemaphoreType.DMA,
            pltpu.SemaphoreType.DMA,
        ],
    )
    pred = run(theta2, alpha2, beta2, sidx, qidx, spar, qpar)
    return pred.reshape(batch, 1)
